# Initial kernel scaffold; baseline (speedup 1.0000x reference)
import jax
import jax.numpy as jnp
from jax.experimental import pallas as pl


def kernel(x, batch, params):
    raise NotImplementedError("write your pallas kernel here")


# speedup vs baseline: 21.7559x; 21.7559x over previous
import jax
import jax.numpy as jnp
from jax.experimental import pallas as pl
from jax.experimental.pallas import tpu as pltpu

N = 10000
NE = 20
KNN = 40
ROWS = 200
CTILE = 500
NT = N // CTILE
_F32_INF_BITS = 0x7F800000
_HI = jax.lax.Precision.HIGHEST


def _dot(a, b):
    return jnp.dot(a, b, preferred_element_type=jnp.float32, precision=_HI)


def _dotd(a, b):
    return jnp.dot(a, b, preferred_element_type=jnp.float32)


def _row(p):
    return p.reshape(1, -1)


def _bn_apply(x, m, v, g, b, eps=1e-5):
    return (x - m) / jnp.sqrt(v + eps) * g + b


def _tanh(x):
    xc = jnp.clip(x, -9.0, 9.0)
    x2 = xc * xc
    num = jnp.float32(-2.76076847742355e-16)
    num = num * x2 + jnp.float32(2.00018790482477e-13)
    num = num * x2 + jnp.float32(-8.60467152213735e-11)
    num = num * x2 + jnp.float32(5.12229709037114e-08)
    num = num * x2 + jnp.float32(1.48572235717979e-05)
    num = num * x2 + jnp.float32(6.37261928875436e-04)
    num = num * x2 + jnp.float32(4.89352455891786e-03)
    num = num * xc
    den = jnp.float32(1.19825839466702e-06)
    den = den * x2 + jnp.float32(1.18534705686654e-04)
    den = den * x2 + jnp.float32(2.26843463243900e-03)
    den = den * x2 + jnp.float32(4.89352518554385e-03)
    return jnp.where(jnp.abs(x) < 0.0004, x, num / den)


def _ge_linear(x, bf, smean, w, b, mn_ref, mx_ref, st_ref):
    f = x.shape[1]
    big = jnp.float32(3.0e38)

    def mbody(e, _):
        m = bf == e.astype(jnp.float32)
        mn_ref[pl.ds(e, 1), :] = jnp.min(jnp.where(m, x, big), axis=0,
                                         keepdims=True)
        mx_ref[pl.ds(e, 1), :] = jnp.max(jnp.where(m, x, -big), axis=0,
                                         keepdims=True)
        return 0

    jax.lax.fori_loop(0, NE, mbody, 0)
    st_ref[:, :] = jnp.concatenate([smean, mn_ref[:, :], mx_ref[:, :]],
                                   axis=1)

    def gbody(e, gath):
        m = bf == e.astype(jnp.float32)
        row = st_ref[pl.ds(e, 1), :]
        return jnp.where(m, row, gath)

    gath = jax.lax.fori_loop(0, NE, gbody,
                             jnp.zeros((x.shape[0], 3 * f), jnp.float32))
    ge = jnp.concatenate([gath, x], axis=1)
    return _dotd(ge, w) + b


def _bn1_body(x_ref, m_ref, v_ref, g_ref, b_ref, out_ref):
    out_ref[:, :] = _bn_apply(x_ref[:, :], m_ref[:, :], v_ref[:, :],
                              g_ref[:, :], b_ref[:, :])


def _gelin_body(x_ref, bf_ref, sm_ref, w_ref, wb_ref, out_ref,
                mn_ref, mx_ref, st_ref):
    out_ref[:, :] = _ge_linear(x_ref[:, :], bf_ref[:, :], sm_ref[:, :],
                               w_ref[:, :], wb_ref[:, :], mn_ref, mx_ref,
                               st_ref)


def _sh_body(x_ref, ws_ref, bs_ref, wh_ref, bh_ref, s_ref, h_ref):
    x = x_ref[:, :]
    s_ref[:, :] = _dotd(x, ws_ref[:, :]) + bs_ref[:, :]
    h_ref[:, :] = _dotd(x, wh_ref[:, :]) + bh_ref[:, :]


def _knn_body(tinfo_ref, srows_ref, brows_ref, st_ref, bt_ref, ht_ref,
              agg_ref, d_ref):
    i = pl.program_id(0)
    t0 = tinfo_ref[i, 0]
    t1 = tinfo_ref[i, 1]
    srows = srows_ref[:, :]
    brows = brows_ref[:, :]
    inf = jnp.float32(jnp.inf)

    def fill(t, _):
        d = jnp.zeros((ROWS, CTILE), jnp.float32)
        for k in range(4):
            diff = srows[:, k:k + 1] - st_ref[t, k:k + 1, :]
            d = d + diff * diff
        same = brows == bt_ref[t, :, :]
        d_ref[t, :, :] = jnp.where(same, d, inf)
        return 0

    jax.lax.fori_loop(t0, t1, fill, 0)

    kf = jnp.float32(KNN)

    def bs_body(_, carry):
        lo, hi = carry
        mid = lo + (hi - lo) // 2
        midf = jax.lax.bitcast_convert_type(mid, jnp.float32)

        def cnt_body(t, acc):
            d = d_ref[t, :, :]
            return acc + jnp.sum((d <= midf).astype(jnp.float32), axis=1,
                                 keepdims=True)

        cnt = jax.lax.fori_loop(t0, t1, cnt_body,
                                jnp.zeros((ROWS, 1), jnp.float32))
        ge = cnt >= kf
        lo = jnp.where(ge, lo, mid + 1)
        hi = jnp.where(ge, mid, hi)
        return lo, hi

    lo0 = jnp.zeros((ROWS, 1), jnp.int32)
    hi0 = jnp.full((ROWS, 1), _F32_INF_BITS, jnp.int32)
    _, hi = jax.lax.fori_loop(0, 31, bs_body, (lo0, hi0))
    thr = jax.lax.bitcast_convert_type(hi, jnp.float32)

    neg = jnp.float32(-3.0e38)

    def agg_body(t, carry):
        macc, mxcols = carry
        d = d_ref[t, :, :]
        sel = d <= thr
        w = jnp.where(sel, jnp.exp(-10.0 * d), 0.0)
        ht = ht_ref[t, :, :]
        macc = macc + jax.lax.dot_general(
            w, ht, (((1,), (1,)), ((), ())),
            preferred_element_type=jnp.float32, precision=_HI)
        new_cols = []
        for f in range(22):
            msg = jnp.where(sel, w * ht[f:f + 1, :], neg)
            new_cols.append(jnp.maximum(mxcols[f],
                                        jnp.max(msg, axis=1, keepdims=True)))
        return macc, tuple(new_cols)

    macc0 = jnp.zeros((ROWS, 22), jnp.float32)
    mx0 = tuple(jnp.full((ROWS, 1), neg, jnp.float32) for _ in range(22))
    macc, mxcols = jax.lax.fori_loop(t0, t1, agg_body, (macc0, mx0))
    agg_ref[:, :] = jnp.concatenate([macc * (1.0 / KNN)] + list(mxcols),
                                    axis=1)


def _u1_body(x_ref, agg_ref, w1_ref, w2_ref, b2_ref, out_ref):
    out_ref[:, :] = _dotd(x_ref[:, :], w1_ref[:, :]) + (
        _dotd(agg_ref[:, :], w2_ref[:, :]) + b2_ref[:, :])


def _u2_body(y_ref, m_ref, v_ref, g_ref, b_ref, w_ref, wb_ref, out_ref):
    y = _bn_apply(y_ref[:, :], m_ref[:, :], v_ref[:, :],
                  g_ref[:, :], b_ref[:, :])
    out_ref[:, :] = _tanh(_dotd(y, w_ref[:, :]) + wb_ref[:, :])


def _u3a_body(t1_ref, m_ref, v_ref, g_ref, b_ref, wp2_ref, bp2_ref,
              out_ref):
    y = _bn_apply(t1_ref[:, :], m_ref[:, :], v_ref[:, :],
                  g_ref[:, :], b_ref[:, :])
    out_ref[:, :] = _tanh(_dotd(y, wp2_ref[:, :]) + bp2_ref[:, :])


def _u3b_body(t2_ref, bf_ref, sm_ref, wo_ref, bo_ref, out_ref,
              mn_ref, mx_ref, st_ref):
    out_ref[:, :] = _tanh(_ge_linear(t2_ref[:, :], bf_ref[:, :],
                                     sm_ref[:, :], wo_ref[:, :],
                                     bo_ref[:, :], mn_ref, mx_ref, st_ref))


def _bnapply_body(y_ref, m_ref, v_ref, g_ref, b_ref, out_ref):
    out_ref[:, :] = _bn_apply(y_ref[:, :], m_ref[:, :], v_ref[:, :],
                              g_ref[:, :], b_ref[:, :])


def _h0_body(f_ref, w_ref, b_ref, out_ref):
    out_ref[:, :] = jnp.maximum(_dotd(f_ref[:, :], w_ref[:, :])
                                + b_ref[:, :], 0.0)


def _hmid_body(z_ref, m_ref, v_ref, g_ref, b_ref, w_ref, wb_ref, out_ref):
    z = _bn_apply(z_ref[:, :], m_ref[:, :], v_ref[:, :],
                  g_ref[:, :], b_ref[:, :])
    out_ref[:, :] = jnp.maximum(_dotd(z, w_ref[:, :]) + wb_ref[:, :], 0.0)


def _htail_body(z_ref, m_ref, v_ref, g_ref, b_ref,
                w1_ref, b1_ref, w2_ref, b2_ref, w3_ref, b3_ref, out_ref):
    z = _bn_apply(z_ref[:, :], m_ref[:, :], v_ref[:, :],
                  g_ref[:, :], b_ref[:, :])
    z = jnp.maximum(_dotd(z, w1_ref[:, :]) + b1_ref[:, :], 0.0)
    z = jnp.maximum(_dotd(z, w2_ref[:, :]) + b2_ref[:, :], 0.0)
    out_ref[:, :] = _dotd(z, w3_ref[:, :]) + b3_ref[:, :]


def _call(body, out_cols, n_scratch_cols=None):
    scratch = []
    if n_scratch_cols is not None:
        scratch = [pltpu.VMEM((NE, n_scratch_cols), jnp.float32),
                   pltpu.VMEM((NE, n_scratch_cols), jnp.float32),
                   pltpu.VMEM((NE, 3 * n_scratch_cols), jnp.float32)]
    return pl.pallas_call(
        body,
        out_shape=jax.ShapeDtypeStruct((N, out_cols), jnp.float32),
        scratch_shapes=scratch,
    )


def _stats(y):
    return (_row(jnp.mean(y, axis=0)), _row(jnp.var(y, axis=0)))


def kernel(x, batch, params):
    batch = batch.astype(jnp.int32)
    bf = batch.astype(jnp.float32).reshape(N, 1)
    p = params

    offs = jnp.searchsorted(batch, jnp.arange(NE + 1, dtype=jnp.int32),
                            side='left').astype(jnp.int32)
    first = batch[jnp.arange(N // ROWS, dtype=jnp.int32) * ROWS]
    last = batch[jnp.arange(1, N // ROWS + 1, dtype=jnp.int32) * ROWS - 1]
    t0 = offs[first] // CTILE
    t1 = (offs[last + 1] + CTILE - 1) // CTILE
    tinfo = jnp.stack([t0, t1], axis=1)

    ones = jnp.ones((N,), jnp.float32)
    cnt = jnp.maximum(jax.ops.segment_sum(ones, batch, num_segments=NE),
                      1.0)[:, None]

    def _smean(y):
        return jax.ops.segment_sum(y, batch, num_segments=NE) / cnt

    m1, v1 = _stats(x)
    xn = _call(_bn1_body, 8)(x, m1, v1, _row(p['bn1']['g']),
                             _row(p['bn1']['b']))
    xc = _call(_gelin_body, 64, 8)(xn, bf, _smean(xn), p['input']['w'],
                                   _row(p['input']['b']))

    knn_call = pl.pallas_call(
        _knn_body,
        grid=(N // ROWS,),
        in_specs=[
            pl.BlockSpec(memory_space=pltpu.SMEM),
            pl.BlockSpec((ROWS, 4), lambda i: (i, 0)),
            pl.BlockSpec((ROWS, 1), lambda i: (i, 0)),
            pl.BlockSpec((NT, 4, CTILE), lambda i: (0, 0, 0)),
            pl.BlockSpec((NT, 1, CTILE), lambda i: (0, 0, 0)),
            pl.BlockSpec((NT, 22, CTILE), lambda i: (0, 0, 0)),
        ],
        out_specs=pl.BlockSpec((ROWS, 44), lambda i: (i, 0)),
        out_shape=jax.ShapeDtypeStruct((N, 44), jnp.float32),
        scratch_shapes=[pltpu.VMEM((NT, ROWS, CTILE), jnp.float32)],
    )

    feats = []
    for blk in p['blocks']:
        s, h = pl.pallas_call(
            _sh_body,
            out_shape=(jax.ShapeDtypeStruct((N, 4), jnp.float32),
                       jax.ShapeDtypeStruct((N, 22), jnp.float32)),
        )(xc, blk['lin_s']['w'], _row(blk['lin_s']['b']),
          blk['lin_h']['w'], _row(blk['lin_h']['b']))
        st = s.T.reshape(4, NT, CTILE).transpose(1, 0, 2)
        bt = bf.T.reshape(1, NT, CTILE).transpose(1, 0, 2)
        ht = h.T.reshape(22, NT, CTILE).transpose(1, 0, 2)
        agg = knn_call(tinfo, s, bf, st, bt, ht)

        ylin = _call(_u1_body, 96)(
            xc, agg, blk['lin_out1']['w'], blk['lin_out2']['w'],
            _row(blk['lin_out2']['b']))
        ma, va = _stats(ylin)
        tt1 = _call(_u2_body, 128)(
            ylin, ma, va, _row(blk['bn_a']['g']), _row(blk['bn_a']['b']),
            blk['pg1']['w'], _row(blk['pg1']['b']))
        mb, vb = _stats(tt1)
        tt2 = _call(_u3a_body, 96)(
            tt1, mb, vb, _row(blk['bn_b']['g']), _row(blk['bn_b']['b']),
            blk['pg2']['w'], _row(blk['pg2']['b']))
        tt3 = _call(_u3b_body, 96, 96)(
            tt2, bf, _smean(tt2), blk['out_lin']['w'],
            _row(blk['out_lin']['b']))
        mc, vc = _stats(tt3)
        xc = _call(_bnapply_body, 96)(
            tt3, mc, vc, _row(blk['bn_c']['g']), _row(blk['bn_c']['b']))
        feats.append(xc)

    fcat = jnp.concatenate(feats, axis=1)
    z = _call(_h0_body, 128)(fcat, p['dense'][0]['lin']['w'],
                             _row(p['dense'][0]['lin']['b']))
    for i in (1, 2, 3):
        md, vd = _stats(z)
        z = _call(_hmid_body, 128)(
            z, md, vd, _row(p['dense'][i - 1]['bn']['g']),
            _row(p['dense'][i - 1]['bn']['b']),
            p['dense'][i]['lin']['w'], _row(p['dense'][i]['lin']['b']))
    md, vd = _stats(z)
    out = _call(_htail_body, 8)(
        z, md, vd, _row(p['dense'][3]['bn']['g']),
        _row(p['dense'][3]['bn']['b']),
        p['out1']['w'], _row(p['out1']['b']),
        p['out2']['w'], _row(p['out2']['b']),
        p['out3']['w'], _row(p['out3']['b']))
    return out
